# baseline (speedup 1.0000x reference)
import functools

import jax
import jax.numpy as jnp
from jax import lax
from jax.experimental import pallas as pl
from jax.experimental.pallas import tpu as pltpu
from jax.experimental.pallas import tpu_sc as plsc

_VOCAB = 1000000
_D = 64
_BATCH = 4096
_SEQ = 200
_TOTAL = _BATCH * _SEQ
_NC = 2
_NS = 16
_NW = _NC * _NS
_PER_W = _TOTAL // _NW
_CHUNK = 128
_NCHUNK = _PER_W // _CHUNK
_SCALE = 8.0

_mesh = plsc.VectorSubcoreMesh(core_axis_name="c", subcore_axis_name="s")


@functools.partial(
    pl.kernel,
    mesh=_mesh,
    out_type=jax.ShapeDtypeStruct((_TOTAL // 2, 128), jnp.float32),
    scratch_types=(
        [pltpu.VMEM((_NCHUNK, _CHUNK), jnp.int32)]
        + [pltpu.VMEM((_CHUNK,), jnp.int32)] * 4
        + [pltpu.VMEM((_CHUNK, 128), jnp.float32)] * 4
        + [pltpu.VMEM((_CHUNK // 2, 128), jnp.float32)] * 2
        + [pltpu.SemaphoreType.DMA] * 6
    ),
    compiler_params=pltpu.CompilerParams(
        use_tc_tiling_on_sc=True, needs_layout_passes=False),
)
def _emb_kernel(idx_hbm, tab_hbm, out_hbm, idx_v,
                f0, f1, f2, f3, g0, g1, g2, g3, o0, o1,
                gs0, gs1, gs2, gs3, ss0, ss1):
    wid = lax.axis_index("s") * _NC + lax.axis_index("c")
    offb = (f0, f1, f2, f3)
    gbufs = (g0, g1, g2, g3)
    obufs = (o0, o1)
    gsems = (gs0, gs1, gs2, gs3)
    ssems = (ss0, ss1)

    pltpu.sync_copy(idx_hbm.at[pl.ds(pl.multiple_of(wid * _NCHUNK, 8), _NCHUNK)], idx_v)
    out_base = wid * _PER_W // 2

    iota16 = lax.iota(jnp.int32, 16)
    zeros16 = jnp.full((16,), 0, jnp.int32)

    def prep_and_fire(s, b):
        for g in range(8):
            ix = idx_v[s, pl.ds(16 * g, 16)]
            offb[b][pl.ds(16 * g, 16)] = (ix & 1) << 6
            idx_v[s, pl.ds(16 * g, 16)] = lax.shift_right_logical(ix, 1)
        pltpu.async_copy(tab_hbm.at[idx_v.at[s]], gbufs[b], gsems[b])

    def wait_in(s, b):
        pltpu.make_async_copy(tab_hbm.at[idx_v.at[s]], gbufs[b], gsems[b]).wait()

    def out_slice(s):
        return out_hbm.at[pl.ds(pl.multiple_of(out_base + s * (_CHUNK // 2), 8), _CHUNK // 2)]

    def fire_out(s, b):
        pltpu.async_copy(obufs[b], out_slice(s), ssems[b])

    def wait_out(s, b):
        pltpu.make_async_copy(obufs[b], out_slice(s), ssems[b]).wait()

    def select_scale(fb, g, o):
        @plsc.parallel_loop(0, _CHUNK // 16, unroll=2)
        def _(qg):
            for ql in range(16):
                q = qg * 16 + ql
                off_q = plsc.load_gather(fb, [zeros16 + q])
                row_q = zeros16 + q
                half = (ql & 1) * 64
                orow = qg * 8 + ql // 2
                for k in range(4):
                    v = plsc.load_gather(g, [row_q, off_q + (16 * k) + iota16])
                    o[orow, pl.ds(half + 16 * k, 16)] = v * _SCALE

    for j in range(3):
        prep_and_fire(j, j)

    def loop(t, carry):
        for b in range(4):
            s = 4 * t + b
            wait_in(s, b)

            @pl.when(s >= 2)
            def _():
                wait_out(s - 2, b & 1)
            select_scale(offb[b], gbufs[b], obufs[b & 1])

            @pl.when(s + 3 < _NCHUNK)
            def _():
                prep_and_fire(s + 3, (b + 3) % 4)
            fire_out(s, b & 1)
        return carry

    lax.fori_loop(0, _NCHUNK // 4, loop, 0)
    wait_out(_NCHUNK - 2, 0)
    wait_out(_NCHUNK - 1, 1)


def kernel(x, input_embedding_table):
    idx = x.reshape(_NW * _NCHUNK, _CHUNK)
    tab2 = input_embedding_table.reshape(_VOCAB // 2, 128)
    out2 = _emb_kernel(idx, tab2)
    return out2.reshape(_BATCH, _SEQ, _D)


# speedup vs baseline: 1.2239x; 1.2239x over previous
import functools

import jax
import jax.numpy as jnp
from jax import lax
from jax.experimental import pallas as pl
from jax.experimental.pallas import tpu as pltpu
from jax.experimental.pallas import tpu_sc as plsc

_VOCAB = 1000000
_D = 64
_BATCH = 4096
_SEQ = 200
_TOTAL = _BATCH * _SEQ
_NC = 2
_NS = 16
_NW = _NC * _NS
_PER_W = _TOTAL // _NW
_CHUNK = 128
_NCHUNK = _PER_W // _CHUNK
_M = 8
_K = 4
_SCALE = 8.0

_mesh = plsc.VectorSubcoreMesh(core_axis_name="c", subcore_axis_name="s")


@functools.partial(
    pl.kernel,
    mesh=_mesh,
    out_type=jax.ShapeDtypeStruct((_TOTAL, _D), jnp.float32),
    scratch_types=(
        [pltpu.VMEM((_NCHUNK, _CHUNK), jnp.int32)]
        + [pltpu.VMEM((_CHUNK, _D), jnp.float32)] * _M
        + [pltpu.SemaphoreType.DMA] * (2 * _M)
    ),
    compiler_params=pltpu.CompilerParams(use_tc_tiling_on_sc=False),
)
def _emb_kernel(idx_hbm, table_hbm, out_hbm, idx_v, *rest):
    bufs = rest[:_M]
    gsem = rest[_M:2 * _M]
    ssem = rest[2 * _M:]

    wid = lax.axis_index("s") * _NC + lax.axis_index("c")
    pltpu.sync_copy(idx_hbm.at[pl.ds(wid * _NCHUNK, _NCHUNK)], idx_v)

    out_base = wid * _PER_W

    def fire_gather(chunk, b):
        pltpu.async_copy(table_hbm.at[idx_v.at[chunk]], bufs[b], gsem[b])

    def wait_gather(chunk, b):
        pltpu.make_async_copy(table_hbm.at[idx_v.at[chunk]], bufs[b], gsem[b]).wait()

    def out_slice(chunk):
        return out_hbm.at[pl.ds(out_base + chunk * _CHUNK, _CHUNK)]

    def fire_store(chunk, b):
        pltpu.async_copy(bufs[b], out_slice(chunk), ssem[b])

    def wait_store(chunk, b):
        pltpu.make_async_copy(bufs[b], out_slice(chunk), ssem[b]).wait()

    def scale_buf(buf):
        def row_body(r, carry):
            for c in range(_D // 16):
                buf[r, pl.ds(c * 16, 16)] = buf[r, pl.ds(c * 16, 16)] * _SCALE
            return carry
        lax.fori_loop(0, _CHUNK, row_body, 0, unroll=4)

    for b in range(_K):
        fire_gather(b, b)

    def body(j, carry):
        for b in range(_M):
            c = j * _M + b
            wait_gather(c, b)
            scale_buf(bufs[b])
            fire_store(c, b)
            f = c + _K
            fb = (b + _K) % _M

            @pl.when(f < _NCHUNK)
            def _():
                @pl.when(c >= _K)
                def _():
                    wait_store(c - _K, fb)
                fire_gather(f, fb)
        return carry

    lax.fori_loop(0, _NCHUNK // _M, body, 0)

    for t in range(2 * _K):
        c = _NCHUNK - 2 * _K + t
        wait_store(c, c % _M)


def kernel(x, input_embedding_table):
    idx = x.reshape(_NW * _NCHUNK, _CHUNK).astype(jnp.int32)
    out = _emb_kernel(idx, input_embedding_table)
    return out.reshape(_BATCH, _SEQ, _D)
